# baseline (speedup 1.0000x reference)
import functools

import jax
import jax.numpy as jnp
from jax import lax
from jax.experimental import pallas as pl
from jax.experimental.pallas import tpu as pltpu
from jax.experimental.pallas import tpu_sc as plsc

DIM = 64
NCORES = 2
NSUB = 16
NW = NCORES * NSUB
CHUNK = 128
MARGIN = 1.0
C = 0.01


@functools.partial(jax.jit, static_argnums=(0,))
def _sc_distances(bcat, ent2, rel2, hh, tt, rr, ho, to, ro):
    per_w = bcat // NW
    n_chunks = per_w // CHUNK
    mesh = plsc.VectorSubcoreMesh(core_axis_name="c", subcore_axis_name="s")

    @functools.partial(
        pl.kernel,
        mesh=mesh,
        compiler_params=pltpu.CompilerParams(use_tc_tiling_on_sc=True),
        out_type=[
            jax.ShapeDtypeStruct((bcat,), jnp.float32),
            jax.ShapeDtypeStruct((NW, 16), jnp.float32),
            jax.ShapeDtypeStruct((NW, 16), jnp.float32),
        ],
        scratch_types=[
            pltpu.VMEM((CHUNK,), jnp.int32),
            pltpu.VMEM((CHUNK,), jnp.int32),
            pltpu.VMEM((CHUNK,), jnp.int32),
            pltpu.VMEM((CHUNK,), jnp.float32),
            pltpu.VMEM((CHUNK,), jnp.float32),
            pltpu.VMEM((CHUNK,), jnp.float32),
            pltpu.VMEM((CHUNK, 2 * DIM), jnp.float32),
            pltpu.VMEM((CHUNK, 2 * DIM), jnp.float32),
            pltpu.VMEM((CHUNK, 2 * DIM), jnp.float32),
            pltpu.VMEM((CHUNK,), jnp.float32),
            pltpu.VMEM((16,), jnp.float32),
            pltpu.SemaphoreType.DMA,
        ],
    )
    def k(ent_hbm, rel_hbm, hh_hbm, tt_hbm, rr_hbm, ho_hbm, to_hbm, ro_hbm,
          dsq_hbm, epen_hbm, rpen_hbm,
          h_v, t_v, r_v, hp_v, tp_v, rp_v, hrow, trow, rrow, dbuf, penbuf,
          sem):
        wid = lax.axis_index("s") * NCORES + lax.axis_index("c")
        base_w = wid * per_w
        lanes = lax.iota(jnp.int32, 16)
        first = lanes == 0

        dnums = lax.GatherDimensionNumbers(
            offset_dims=(), collapsed_slice_dims=(0,), start_index_map=(0,))

        def shuf(x, idx):
            return lax.gather(
                x, idx[:, None], dimension_numbers=dnums, slice_sizes=(1,),
                mode=lax.GatherScatterMode.PROMISE_IN_BOUNDS)

        def xsum(x):
            for s in (8, 4, 2, 1):
                x = x + shuf(x, lanes ^ s)
            return x

        def chunk_body(ci, accs):
            base = base_w + ci * CHUNK
            pltpu.sync_copy(hh_hbm.at[pl.ds(base, CHUNK)], h_v)
            pltpu.sync_copy(tt_hbm.at[pl.ds(base, CHUNK)], t_v)
            pltpu.sync_copy(rr_hbm.at[pl.ds(base, CHUNK)], r_v)
            pltpu.sync_copy(ho_hbm.at[pl.ds(base, CHUNK)], hp_v)
            pltpu.sync_copy(to_hbm.at[pl.ds(base, CHUNK)], tp_v)
            pltpu.sync_copy(ro_hbm.at[pl.ds(base, CHUNK)], rp_v)
            c1 = pltpu.async_copy(ent_hbm.at[h_v], hrow, sem)
            c2 = pltpu.async_copy(ent_hbm.at[t_v], trow, sem)
            c3 = pltpu.async_copy(rel_hbm.at[r_v], rrow, sem)
            c1.wait()
            c2.wait()
            c3.wait()

            def group_body(g, carry):
                ea, ra = carry
                acc_d = jnp.zeros((16,), jnp.float32)
                pv_h = hp_v[pl.ds(g * 16, 16)]
                pv_t = tp_v[pl.ds(g * 16, 16)]
                pv_r = rp_v[pl.ds(g * 16, 16)]
                for jj in range(16):
                    j = g * 16 + jj
                    bidx = jnp.full((16,), jj, jnp.int32)
                    ph = shuf(pv_h, bidx)
                    pt = shuf(pv_t, bidx)
                    pr = shuf(pv_r, bidx)
                    sd = sh = st = sr = None
                    for q in range(DIM // 16):
                        hlo = hrow[j, pl.ds(q * 16, 16)]
                        hq = hlo + ph * (hrow[j, pl.ds(DIM + q * 16, 16)]
                                         - hlo)
                        rlo = rrow[j, pl.ds(q * 16, 16)]
                        rq = rlo + pr * (rrow[j, pl.ds(DIM + q * 16, 16)]
                                         - rlo)
                        tlo = trow[j, pl.ds(q * 16, 16)]
                        tq = tlo + pt * (trow[j, pl.ds(DIM + q * 16, 16)]
                                         - tlo)
                        d = hq + rq - tq
                        if q == 0:
                            sd, sh, st, sr = d * d, hq * hq, tq * tq, rq * rq
                        else:
                            sd = sd + d * d
                            sh = sh + hq * hq
                            st = st + tq * tq
                            sr = sr + rq * rq
                    csd = xsum(sd)
                    csh = xsum(sh)
                    cst = xsum(st)
                    csr = xsum(sr)
                    acc_d = jnp.where(lanes == jj, csd, acc_d)
                    ea = ea + jnp.where(
                        first,
                        jnp.maximum(csh - 1.0, 0.0)
                        + jnp.maximum(cst - 1.0, 0.0),
                        0.0)
                    ra = ra + jnp.where(
                        first, jnp.maximum(csr - 1.0, 0.0), 0.0)
                dbuf[pl.ds(g * 16, 16)] = acc_d
                return (ea, ra)

            accs = lax.fori_loop(0, CHUNK // 16, group_body, accs)
            pltpu.sync_copy(dbuf, dsq_hbm.at[pl.ds(base, CHUNK)])
            return accs

        zero = jnp.zeros((16,), jnp.float32)
        ent_acc, rel_acc = lax.fori_loop(0, n_chunks, chunk_body, (zero, zero))
        penbuf[...] = ent_acc
        pltpu.sync_copy(penbuf, epen_hbm.at[wid])
        penbuf[...] = rel_acc
        pltpu.sync_copy(penbuf, rpen_hbm.at[wid])

    return k(ent2, rel2, hh, tt, rr, ho, to, ro)


PACK_W = 1024


def _pack_pairs(table_t):
    d, n = table_t.shape
    w = PACK_W
    hw = w // 2
    grid = (n + w - 1) // w

    def body(in_ref, out_ref):
        x = in_ref[...]
        out_ref[...] = jnp.concatenate(
            [x[:, 0:hw].T, x[:, hw:w].T], axis=1)

    return pl.pallas_call(
        body,
        grid=(grid,),
        in_specs=[pl.BlockSpec((d, w), lambda i: (0, i))],
        out_specs=pl.BlockSpec((hw, 2 * d), lambda i: (i, 0)),
        out_shape=jax.ShapeDtypeStruct((grid * hw, 2 * d), jnp.float32),
    )(table_t)


def _pair_split(e):
    hw = PACK_W // 2
    blk = e // PACK_W
    off = e % PACK_W
    return blk * hw + off % hw, (off // hw).astype(jnp.float32)


def _finalize(pos_sq, neg_sq, epen, rpen):
    batch = pos_sq.shape[0] * pos_sq.shape[1]

    def body(pos_ref, neg_ref, epen_ref, rpen_ref, out_ref):
        p = jnp.sqrt(pos_ref[...])
        n = jnp.sqrt(neg_ref[...])
        loss = jnp.sum(jnp.maximum(p - n + MARGIN, 0.0)) / batch
        ent = jnp.sum(epen_ref[...]) / (4.0 * batch)
        rel = jnp.sum(rpen_ref[...]) / (2.0 * batch)
        out_ref[...] = jnp.full((1, 1), loss + C * (ent + rel), jnp.float32)

    return pl.pallas_call(
        body,
        out_shape=jax.ShapeDtypeStruct((1, 1), jnp.float32),
    )(pos_sq, neg_sq, epen, rpen)


def kernel(triple, corrupted_triple, entity_emb, relation_emb):
    h = triple[:, 0].astype(jnp.int32)
    r = triple[:, 1].astype(jnp.int32)
    t = triple[:, 2].astype(jnp.int32)
    hc = corrupted_triple[:, 0].astype(jnp.int32)
    rc = corrupted_triple[:, 1].astype(jnp.int32)
    tc = corrupted_triple[:, 2].astype(jnp.int32)
    batch = h.shape[0]
    hh = jnp.concatenate([h, hc])
    tt = jnp.concatenate([t, tc])
    rr = jnp.concatenate([r, rc])
    ent2 = _pack_pairs(entity_emb.T)
    rel2 = _pack_pairs(relation_emb.T)
    hh2, ho = _pair_split(hh)
    tt2, to = _pair_split(tt)
    rr2, ro = _pair_split(rr)
    dsq, epen, rpen = _sc_distances(
        2 * batch, ent2, rel2, hh2, tt2, rr2, ho, to, ro)
    pos_sq = dsq[:batch].reshape(128, -1)
    neg_sq = dsq[batch:].reshape(128, -1)
    out = _finalize(pos_sq, neg_sq, epen, rpen)
    return out[0, 0]


# speedup vs baseline: 2.2308x; 2.2308x over previous
import functools

import jax
import jax.numpy as jnp
from jax import lax
from jax.experimental import pallas as pl
from jax.experimental.pallas import tpu as pltpu
from jax.experimental.pallas import tpu_sc as plsc

DIM = 64
NCORES = 2
NSUB = 16
NW = NCORES * NSUB
CHUNK = 128
MARGIN = 1.0
C = 0.01


@functools.partial(jax.jit, static_argnums=(0,))
def _sc_distances(bcat, ent2, rel2, hh, tt, rr, ho, to, ro):
    per_w = bcat // NW
    n_chunks = per_w // CHUNK
    mesh = plsc.VectorSubcoreMesh(core_axis_name="c", subcore_axis_name="s")

    @functools.partial(
        pl.kernel,
        mesh=mesh,
        compiler_params=pltpu.CompilerParams(use_tc_tiling_on_sc=True),
        out_type=[
            jax.ShapeDtypeStruct((bcat,), jnp.float32),
            jax.ShapeDtypeStruct((NW, 16), jnp.float32),
            jax.ShapeDtypeStruct((NW, 16), jnp.float32),
        ],
        scratch_types=[
            pltpu.VMEM((CHUNK,), jnp.int32),
            pltpu.VMEM((CHUNK,), jnp.int32),
            pltpu.VMEM((CHUNK,), jnp.int32),
            pltpu.VMEM((CHUNK,), jnp.float32),
            pltpu.VMEM((CHUNK,), jnp.float32),
            pltpu.VMEM((CHUNK,), jnp.float32),
            pltpu.VMEM((CHUNK, 2 * DIM), jnp.float32),
            pltpu.VMEM((CHUNK, 2 * DIM), jnp.float32),
            pltpu.VMEM((CHUNK, 2 * DIM), jnp.float32),
            pltpu.VMEM((CHUNK,), jnp.float32),
            pltpu.VMEM((16,), jnp.float32),
            pltpu.SemaphoreType.DMA,
        ],
    )
    def k(ent_hbm, rel_hbm, hh_hbm, tt_hbm, rr_hbm, ho_hbm, to_hbm, ro_hbm,
          dsq_hbm, epen_hbm, rpen_hbm,
          h_v, t_v, r_v, hp_v, tp_v, rp_v, hrow, trow, rrow, dbuf, penbuf,
          sem):
        wid = lax.axis_index("s") * NCORES + lax.axis_index("c")
        base_w = wid * per_w
        lanes = lax.iota(jnp.int32, 16)
        first = lanes == 0

        dnums = lax.GatherDimensionNumbers(
            offset_dims=(), collapsed_slice_dims=(0,), start_index_map=(0,))

        def shuf(x, idx):
            return lax.gather(
                x, idx[:, None], dimension_numbers=dnums, slice_sizes=(1,),
                mode=lax.GatherScatterMode.PROMISE_IN_BOUNDS)

        def xsum(x):
            for s in (8, 4, 2, 1):
                x = x + shuf(x, lanes ^ s)
            return x

        def chunk_body(ci, accs):
            base = base_w + ci * CHUNK
            pltpu.sync_copy(hh_hbm.at[pl.ds(base, CHUNK)], h_v)
            pltpu.sync_copy(tt_hbm.at[pl.ds(base, CHUNK)], t_v)
            pltpu.sync_copy(rr_hbm.at[pl.ds(base, CHUNK)], r_v)
            pltpu.sync_copy(ho_hbm.at[pl.ds(base, CHUNK)], hp_v)
            pltpu.sync_copy(to_hbm.at[pl.ds(base, CHUNK)], tp_v)
            pltpu.sync_copy(ro_hbm.at[pl.ds(base, CHUNK)], rp_v)
            c1 = pltpu.async_copy(ent_hbm.at[h_v], hrow, sem)
            c2 = pltpu.async_copy(ent_hbm.at[t_v], trow, sem)
            c3 = pltpu.async_copy(rel_hbm.at[r_v], rrow, sem)
            c1.wait()
            c2.wait()
            c3.wait()

            def group_body(g, carry):
                ea, ra = carry
                acc_d = jnp.zeros((16,), jnp.float32)
                pv_h = hp_v[pl.ds(g * 16, 16)]
                pv_t = tp_v[pl.ds(g * 16, 16)]
                pv_r = rp_v[pl.ds(g * 16, 16)]
                for jj in range(16):
                    j = g * 16 + jj
                    bidx = jnp.full((16,), jj, jnp.int32)
                    ph = shuf(pv_h, bidx)
                    pt = shuf(pv_t, bidx)
                    pr = shuf(pv_r, bidx)
                    sd = sh = st = sr = None
                    for q in range(DIM // 16):
                        hlo = hrow[j, pl.ds(q * 16, 16)]
                        hq = hlo + ph * (hrow[j, pl.ds(DIM + q * 16, 16)]
                                         - hlo)
                        rlo = rrow[j, pl.ds(q * 16, 16)]
                        rq = rlo + pr * (rrow[j, pl.ds(DIM + q * 16, 16)]
                                         - rlo)
                        tlo = trow[j, pl.ds(q * 16, 16)]
                        tq = tlo + pt * (trow[j, pl.ds(DIM + q * 16, 16)]
                                         - tlo)
                        d = hq + rq - tq
                        if q == 0:
                            sd, sh, st, sr = d * d, hq * hq, tq * tq, rq * rq
                        else:
                            sd = sd + d * d
                            sh = sh + hq * hq
                            st = st + tq * tq
                            sr = sr + rq * rq
                    csd = xsum(sd)
                    csh = xsum(sh)
                    cst = xsum(st)
                    csr = xsum(sr)
                    acc_d = jnp.where(lanes == jj, csd, acc_d)
                    ea = ea + jnp.where(
                        first,
                        jnp.maximum(csh - 1.0, 0.0)
                        + jnp.maximum(cst - 1.0, 0.0),
                        0.0)
                    ra = ra + jnp.where(
                        first, jnp.maximum(csr - 1.0, 0.0), 0.0)
                dbuf[pl.ds(g * 16, 16)] = acc_d
                return (ea, ra)

            accs = lax.fori_loop(0, CHUNK // 16, group_body, accs)
            pltpu.sync_copy(dbuf, dsq_hbm.at[pl.ds(base, CHUNK)])
            return accs

        zero = jnp.zeros((16,), jnp.float32)
        ent_acc, rel_acc = lax.fori_loop(0, n_chunks, chunk_body, (zero, zero))
        penbuf[...] = ent_acc
        pltpu.sync_copy(penbuf, epen_hbm.at[wid])
        penbuf[...] = rel_acc
        pltpu.sync_copy(penbuf, rpen_hbm.at[wid])

    return k(ent2, rel2, hh, tt, rr, ho, to, ro)


PACK_W_ENT = 8192
PACK_W_REL = 1024


def _pack_pairs(table_t, w):
    d, n = table_t.shape
    hw = w // 2
    grid = (n + w - 1) // w

    def body(in_ref, out_ref):
        x = in_ref[...]
        out_ref[...] = jnp.concatenate(
            [x[:, 0:hw].T, x[:, hw:w].T], axis=1)

    return pl.pallas_call(
        body,
        grid=(grid,),
        in_specs=[pl.BlockSpec((d, w), lambda i: (0, i))],
        out_specs=pl.BlockSpec((hw, 2 * d), lambda i: (i, 0)),
        out_shape=jax.ShapeDtypeStruct((grid * hw, 2 * d), jnp.float32),
    )(table_t)


def _pair_split(e, w):
    hw = w // 2
    blk = e // w
    off = e % w
    return blk * hw + off % hw, (off // hw).astype(jnp.float32)


def _finalize(pos_sq, neg_sq, epen, rpen):
    batch = pos_sq.shape[0] * pos_sq.shape[1]

    def body(pos_ref, neg_ref, epen_ref, rpen_ref, out_ref):
        p = jnp.sqrt(pos_ref[...])
        n = jnp.sqrt(neg_ref[...])
        loss = jnp.sum(jnp.maximum(p - n + MARGIN, 0.0)) / batch
        ent = jnp.sum(epen_ref[...]) / (4.0 * batch)
        rel = jnp.sum(rpen_ref[...]) / (2.0 * batch)
        out_ref[...] = jnp.full((1, 1), loss + C * (ent + rel), jnp.float32)

    return pl.pallas_call(
        body,
        out_shape=jax.ShapeDtypeStruct((1, 1), jnp.float32),
    )(pos_sq, neg_sq, epen, rpen)


def kernel(triple, corrupted_triple, entity_emb, relation_emb):
    h = triple[:, 0].astype(jnp.int32)
    r = triple[:, 1].astype(jnp.int32)
    t = triple[:, 2].astype(jnp.int32)
    hc = corrupted_triple[:, 0].astype(jnp.int32)
    rc = corrupted_triple[:, 1].astype(jnp.int32)
    tc = corrupted_triple[:, 2].astype(jnp.int32)
    batch = h.shape[0]
    hh = jnp.concatenate([h, hc])
    tt = jnp.concatenate([t, tc])
    rr = jnp.concatenate([r, rc])
    ent2 = _pack_pairs(entity_emb.T, PACK_W_ENT)
    rel2 = _pack_pairs(relation_emb.T, PACK_W_REL)
    hh2, ho = _pair_split(hh, PACK_W_ENT)
    tt2, to = _pair_split(tt, PACK_W_ENT)
    rr2, ro = _pair_split(rr, PACK_W_REL)
    dsq, epen, rpen = _sc_distances(
        2 * batch, ent2, rel2, hh2, tt2, rr2, ho, to, ro)
    pos_sq = dsq[:batch].reshape(128, -1)
    neg_sq = dsq[batch:].reshape(128, -1)
    out = _finalize(pos_sq, neg_sq, epen, rpen)
    return out[0, 0]


# speedup vs baseline: 2.6905x; 1.2061x over previous
import functools

import jax
import jax.numpy as jnp
from jax import lax
from jax.experimental import pallas as pl
from jax.experimental.pallas import tpu as pltpu
from jax.experimental.pallas import tpu_sc as plsc

DIM = 64
NCORES = 2
NSUB = 16
NW = NCORES * NSUB
CHUNK = 128
MARGIN = 1.0
C = 0.01


@functools.partial(jax.jit, static_argnums=(0,))
def _sc_distances(bcat, ent2, rel2, hh, tt, rr, ho, to, ro):
    per_w = bcat // NW
    n_chunks = per_w // CHUNK
    mesh = plsc.VectorSubcoreMesh(core_axis_name="c", subcore_axis_name="s")

    @functools.partial(
        pl.kernel,
        mesh=mesh,
        compiler_params=pltpu.CompilerParams(use_tc_tiling_on_sc=True),
        out_type=[
            jax.ShapeDtypeStruct((bcat,), jnp.float32),
            jax.ShapeDtypeStruct((NW, 16), jnp.float32),
            jax.ShapeDtypeStruct((NW, 16), jnp.float32),
        ],
        scratch_types=[
            pltpu.VMEM((CHUNK,), jnp.int32),
            pltpu.VMEM((CHUNK,), jnp.int32),
            pltpu.VMEM((CHUNK,), jnp.int32),
            pltpu.VMEM((CHUNK,), jnp.float32),
            pltpu.VMEM((CHUNK,), jnp.float32),
            pltpu.VMEM((CHUNK,), jnp.float32),
            pltpu.VMEM((CHUNK, 2 * DIM), jnp.float32),
            pltpu.VMEM((CHUNK, 2 * DIM), jnp.float32),
            pltpu.VMEM((CHUNK, 2 * DIM), jnp.float32),
            pltpu.VMEM((CHUNK,), jnp.float32),
            pltpu.VMEM((16,), jnp.float32),
            pltpu.SemaphoreType.DMA,
        ],
    )
    def k(ent_hbm, rel_hbm, hh_hbm, tt_hbm, rr_hbm, ho_hbm, to_hbm, ro_hbm,
          dsq_hbm, epen_hbm, rpen_hbm,
          h_v, t_v, r_v, hp_v, tp_v, rp_v, hrow, trow, rrow, dbuf, penbuf,
          sem):
        wid = lax.axis_index("s") * NCORES + lax.axis_index("c")
        base_w = wid * per_w
        lanes = lax.iota(jnp.int32, 16)
        first = lanes == 0

        dnums = lax.GatherDimensionNumbers(
            offset_dims=(), collapsed_slice_dims=(0,), start_index_map=(0,))

        def shuf(x, idx):
            return lax.gather(
                x, idx[:, None], dimension_numbers=dnums, slice_sizes=(1,),
                mode=lax.GatherScatterMode.PROMISE_IN_BOUNDS)

        def xsum(x):
            for s in (8, 4, 2, 1):
                x = x + shuf(x, lanes ^ s)
            return x

        def chunk_body(ci, accs):
            base = base_w + ci * CHUNK
            pltpu.sync_copy(hh_hbm.at[pl.ds(base, CHUNK)], h_v)
            pltpu.sync_copy(tt_hbm.at[pl.ds(base, CHUNK)], t_v)
            pltpu.sync_copy(rr_hbm.at[pl.ds(base, CHUNK)], r_v)
            pltpu.sync_copy(ho_hbm.at[pl.ds(base, CHUNK)], hp_v)
            pltpu.sync_copy(to_hbm.at[pl.ds(base, CHUNK)], tp_v)
            pltpu.sync_copy(ro_hbm.at[pl.ds(base, CHUNK)], rp_v)
            c1 = pltpu.async_copy(ent_hbm.at[h_v], hrow, sem)
            c2 = pltpu.async_copy(ent_hbm.at[t_v], trow, sem)
            c3 = pltpu.async_copy(rel_hbm.at[r_v], rrow, sem)
            c1.wait()
            c2.wait()
            c3.wait()

            def group_body(g, carry):
                ea, ra = carry
                acc_d = jnp.zeros((16,), jnp.float32)
                pv_h = hp_v[pl.ds(g * 16, 16)]
                pv_t = tp_v[pl.ds(g * 16, 16)]
                pv_r = rp_v[pl.ds(g * 16, 16)]
                for jj in range(16):
                    j = g * 16 + jj
                    bidx = jnp.full((16,), jj, jnp.int32)
                    ph = shuf(pv_h, bidx)
                    pt = shuf(pv_t, bidx)
                    pr = shuf(pv_r, bidx)
                    sd = sh = st = sr = None
                    for q in range(DIM // 16):
                        hlo = hrow[j, pl.ds(q * 16, 16)]
                        hq = hlo + ph * (hrow[j, pl.ds(DIM + q * 16, 16)]
                                         - hlo)
                        rlo = rrow[j, pl.ds(q * 16, 16)]
                        rq = rlo + pr * (rrow[j, pl.ds(DIM + q * 16, 16)]
                                         - rlo)
                        tlo = trow[j, pl.ds(q * 16, 16)]
                        tq = tlo + pt * (trow[j, pl.ds(DIM + q * 16, 16)]
                                         - tlo)
                        d = hq + rq - tq
                        if q == 0:
                            sd, sh, st, sr = d * d, hq * hq, tq * tq, rq * rq
                        else:
                            sd = sd + d * d
                            sh = sh + hq * hq
                            st = st + tq * tq
                            sr = sr + rq * rq
                    csd = xsum(sd)
                    csh = xsum(sh)
                    cst = xsum(st)
                    csr = xsum(sr)
                    acc_d = jnp.where(lanes == jj, csd, acc_d)
                    ea = ea + jnp.where(
                        first,
                        jnp.maximum(csh - 1.0, 0.0)
                        + jnp.maximum(cst - 1.0, 0.0),
                        0.0)
                    ra = ra + jnp.where(
                        first, jnp.maximum(csr - 1.0, 0.0), 0.0)
                dbuf[pl.ds(g * 16, 16)] = acc_d
                return (ea, ra)

            accs = lax.fori_loop(0, CHUNK // 16, group_body, accs)
            pltpu.sync_copy(dbuf, dsq_hbm.at[pl.ds(base, CHUNK)])
            return accs

        zero = jnp.zeros((16,), jnp.float32)
        ent_acc, rel_acc = lax.fori_loop(0, n_chunks, chunk_body, (zero, zero))
        penbuf[...] = ent_acc
        pltpu.sync_copy(penbuf, epen_hbm.at[wid])
        penbuf[...] = rel_acc
        pltpu.sync_copy(penbuf, rpen_hbm.at[wid])

    return k(ent2, rel2, hh, tt, rr, ho, to, ro)


PACK_W_ENT = 8192
PACK_W_REL = 1024


def _pack_pairs(table_t, w):
    d, n = table_t.shape
    hw = w // 2
    grid = (n + w - 1) // w

    def body(in_ref, out_ref):
        x = in_ref[...]
        out_ref[...] = jnp.concatenate(
            [x[:, 0:hw], x[:, hw:w]], axis=0).T

    return pl.pallas_call(
        body,
        grid=(grid,),
        in_specs=[pl.BlockSpec((d, w), lambda i: (0, i))],
        out_specs=pl.BlockSpec((hw, 2 * d), lambda i: (i, 0)),
        out_shape=jax.ShapeDtypeStruct((grid * hw, 2 * d), jnp.float32),
    )(table_t)


def _pair_split(e, w):
    hw = w // 2
    blk = e // w
    off = e % w
    return blk * hw + off % hw, (off // hw).astype(jnp.float32)


def _finalize(pos_sq, neg_sq, epen, rpen):
    batch = pos_sq.shape[0] * pos_sq.shape[1]

    def body(pos_ref, neg_ref, epen_ref, rpen_ref, out_ref):
        p = jnp.sqrt(pos_ref[...])
        n = jnp.sqrt(neg_ref[...])
        loss = jnp.sum(jnp.maximum(p - n + MARGIN, 0.0)) / batch
        ent = jnp.sum(epen_ref[...]) / (4.0 * batch)
        rel = jnp.sum(rpen_ref[...]) / (2.0 * batch)
        out_ref[...] = jnp.full((1, 1), loss + C * (ent + rel), jnp.float32)

    return pl.pallas_call(
        body,
        out_shape=jax.ShapeDtypeStruct((1, 1), jnp.float32),
    )(pos_sq, neg_sq, epen, rpen)


def kernel(triple, corrupted_triple, entity_emb, relation_emb):
    h = triple[:, 0].astype(jnp.int32)
    r = triple[:, 1].astype(jnp.int32)
    t = triple[:, 2].astype(jnp.int32)
    hc = corrupted_triple[:, 0].astype(jnp.int32)
    rc = corrupted_triple[:, 1].astype(jnp.int32)
    tc = corrupted_triple[:, 2].astype(jnp.int32)
    batch = h.shape[0]
    hh = jnp.concatenate([h, hc])
    tt = jnp.concatenate([t, tc])
    rr = jnp.concatenate([r, rc])
    ent2 = _pack_pairs(entity_emb.T, PACK_W_ENT)
    rel2 = _pack_pairs(relation_emb.T, PACK_W_REL)
    hh2, ho = _pair_split(hh, PACK_W_ENT)
    tt2, to = _pair_split(tt, PACK_W_ENT)
    rr2, ro = _pair_split(rr, PACK_W_REL)
    dsq, epen, rpen = _sc_distances(
        2 * batch, ent2, rel2, hh2, tt2, rr2, ho, to, ro)
    pos_sq = dsq[:batch].reshape(128, -1)
    neg_sq = dsq[batch:].reshape(128, -1)
    out = _finalize(pos_sq, neg_sq, epen, rpen)
    return out[0, 0]


# speedup vs baseline: 3.2442x; 1.2058x over previous
import functools

import jax
import jax.numpy as jnp
from jax import lax
from jax.experimental import pallas as pl
from jax.experimental.pallas import tpu as pltpu
from jax.experimental.pallas import tpu_sc as plsc

DIM = 64
NCORES = 2
NSUB = 16
NW = NCORES * NSUB
CHUNK = 128
MARGIN = 1.0
C = 0.01


@functools.partial(jax.jit, static_argnums=(0,))
def _sc_distances(bcat, ent2, rel2, hh, tt, rr, ho, to, ro):
    per_w = bcat // NW
    n_chunks = per_w // CHUNK
    mesh = plsc.VectorSubcoreMesh(core_axis_name="c", subcore_axis_name="s")

    @functools.partial(
        pl.kernel,
        mesh=mesh,
        compiler_params=pltpu.CompilerParams(use_tc_tiling_on_sc=True),
        out_type=[
            jax.ShapeDtypeStruct((bcat,), jnp.float32),
            jax.ShapeDtypeStruct((NW, 16), jnp.float32),
            jax.ShapeDtypeStruct((NW, 16), jnp.float32),
        ],
        scratch_types=(
            2 * [pltpu.VMEM((CHUNK,), jnp.int32)] * 3
            + 2 * [pltpu.VMEM((CHUNK,), jnp.float32)] * 3
            + 2 * [pltpu.VMEM((CHUNK, 2 * DIM), jnp.float32)] * 3
            + [
                pltpu.VMEM((CHUNK,), jnp.float32),
                pltpu.VMEM((16,), jnp.float32),
                pltpu.SemaphoreType.DMA,
                pltpu.SemaphoreType.DMA,
            ]
        ),
    )
    def k(ent_hbm, rel_hbm, hh_hbm, tt_hbm, rr_hbm, ho_hbm, to_hbm, ro_hbm,
          dsq_hbm, epen_hbm, rpen_hbm,
          h_v0, t_v0, r_v0, h_v1, t_v1, r_v1,
          hp_v0, tp_v0, rp_v0, hp_v1, tp_v1, rp_v1,
          hrow0, trow0, rrow0, hrow1, trow1, rrow1,
          dbuf, penbuf, sem0, sem1):
        wid = lax.axis_index("s") * NCORES + lax.axis_index("c")
        base_w = wid * per_w
        lanes = lax.iota(jnp.int32, 16)
        bufs = [
            (h_v0, t_v0, r_v0, hp_v0, tp_v0, rp_v0, hrow0, trow0, rrow0,
             sem0),
            (h_v1, t_v1, r_v1, hp_v1, tp_v1, rp_v1, hrow1, trow1, rrow1,
             sem1),
        ]

        dnums = lax.GatherDimensionNumbers(
            offset_dims=(), collapsed_slice_dims=(0,), start_index_map=(0,))

        def shuf(x, idx):
            return lax.gather(
                x, idx[:, None], dimension_numbers=dnums, slice_sizes=(1,),
                mode=lax.GatherScatterMode.PROMISE_IN_BOUNDS)

        def issue(ci, buf):
            h_v, t_v, r_v, hp_v, tp_v, rp_v, hrow, trow, rrow, sem = buf
            base = base_w + ci * CHUNK
            pltpu.sync_copy(hh_hbm.at[pl.ds(base, CHUNK)], h_v)
            pltpu.sync_copy(tt_hbm.at[pl.ds(base, CHUNK)], t_v)
            pltpu.sync_copy(rr_hbm.at[pl.ds(base, CHUNK)], r_v)
            pltpu.sync_copy(ho_hbm.at[pl.ds(base, CHUNK)], hp_v)
            pltpu.sync_copy(to_hbm.at[pl.ds(base, CHUNK)], tp_v)
            pltpu.sync_copy(ro_hbm.at[pl.ds(base, CHUNK)], rp_v)
            pltpu.async_copy(ent_hbm.at[h_v], hrow, sem)
            pltpu.async_copy(ent_hbm.at[t_v], trow, sem)
            pltpu.async_copy(rel_hbm.at[r_v], rrow, sem)

        def wait(buf):
            h_v, t_v, r_v, hp_v, tp_v, rp_v, hrow, trow, rrow, sem = buf
            pltpu.make_async_copy(ent_hbm.at[h_v], hrow, sem).wait()
            pltpu.make_async_copy(ent_hbm.at[t_v], trow, sem).wait()
            pltpu.make_async_copy(rel_hbm.at[r_v], rrow, sem).wait()

        def compute(ci, buf, accs):
            h_v, t_v, r_v, hp_v, tp_v, rp_v, hrow, trow, rrow, sem = buf
            base = base_w + ci * CHUNK

            def group_body(g, carry):
                ea, ra = carry
                acc_d = jnp.zeros((16,), jnp.float32)
                pv_h = hp_v[pl.ds(g * 16, 16)]
                pv_t = tp_v[pl.ds(g * 16, 16)]
                pv_r = rp_v[pl.ds(g * 16, 16)]
                for jj in range(16):
                    j = g * 16 + jj
                    bidx = jnp.full((16,), jj, jnp.int32)
                    ph = shuf(pv_h, bidx)
                    pt = shuf(pv_t, bidx)
                    pr = shuf(pv_r, bidx)
                    sd = sh = st = sr = None
                    for q in range(DIM // 16):
                        hlo = hrow[j, pl.ds(q * 16, 16)]
                        hq = hlo + ph * (hrow[j, pl.ds(DIM + q * 16, 16)]
                                         - hlo)
                        rlo = rrow[j, pl.ds(q * 16, 16)]
                        rq = rlo + pr * (rrow[j, pl.ds(DIM + q * 16, 16)]
                                         - rlo)
                        tlo = trow[j, pl.ds(q * 16, 16)]
                        tq = tlo + pt * (trow[j, pl.ds(DIM + q * 16, 16)]
                                         - tlo)
                        d = hq + rq - tq
                        if q == 0:
                            sd, sh, st, sr = d * d, hq * hq, tq * tq, rq * rq
                        else:
                            sd = sd + d * d
                            sh = sh + hq * hq
                            st = st + tq * tq
                            sr = sr + rq * rq
                    csd = sd
                    for s in (8, 4, 2, 1):
                        csd = csd + shuf(csd, lanes ^ s)
                    for s in (8, 4):
                        sh = sh + shuf(sh, lanes ^ s)
                        st = st + shuf(st, lanes ^ s)
                        sr = sr + shuf(sr, lanes ^ s)
                    m = jnp.where(lanes < 4, sh,
                                  jnp.where(lanes < 8, st,
                                            jnp.where(lanes < 12, sr, 0.0)))
                    for s in (2, 1):
                        m = m + shuf(m, lanes ^ s)
                    pe = jnp.maximum(m - 1.0, 0.0)
                    acc_d = jnp.where(lanes == jj, csd, acc_d)
                    ea = ea + jnp.where(lanes < 8, pe, 0.0)
                    ra = ra + jnp.where((lanes >= 8) & (lanes < 12), pe, 0.0)
                dbuf[pl.ds(g * 16, 16)] = acc_d
                return (ea, ra)

            accs = lax.fori_loop(0, CHUNK // 16, group_body, accs)
            pltpu.sync_copy(dbuf, dsq_hbm.at[pl.ds(base, CHUNK)])
            return accs

        issue(0, bufs[0])
        n_super = n_chunks // 2

        def super_body(s, accs):
            c0 = 2 * s
            issue(c0 + 1, bufs[1])
            wait(bufs[0])
            accs = compute(c0, bufs[0], accs)

            @pl.when(s < n_super - 1)
            def _():
                issue(c0 + 2, bufs[0])

            wait(bufs[1])
            return compute(c0 + 1, bufs[1], accs)

        zero = jnp.zeros((16,), jnp.float32)
        ent_acc, rel_acc = lax.fori_loop(0, n_super, super_body, (zero, zero))
        penbuf[...] = ent_acc * 0.25
        pltpu.sync_copy(penbuf, epen_hbm.at[wid])
        penbuf[...] = rel_acc * 0.25
        pltpu.sync_copy(penbuf, rpen_hbm.at[wid])

    return k(ent2, rel2, hh, tt, rr, ho, to, ro)


PACK_W_ENT = 16384
PACK_W_REL = 1024


def _pack_pairs(table_t, w):
    d, n = table_t.shape
    hw = w // 2
    grid = (n + w - 1) // w

    def body(in_ref, out_ref):
        x = in_ref[...]
        out_ref[...] = jnp.concatenate(
            [x[:, 0:hw], x[:, hw:w]], axis=0).T

    return pl.pallas_call(
        body,
        grid=(grid,),
        in_specs=[pl.BlockSpec((d, w), lambda i: (0, i))],
        out_specs=pl.BlockSpec((hw, 2 * d), lambda i: (i, 0)),
        out_shape=jax.ShapeDtypeStruct((grid * hw, 2 * d), jnp.float32),
    )(table_t)


def _pair_split(e, w):
    hw = w // 2
    blk = e // w
    off = e % w
    return blk * hw + off % hw, (off // hw).astype(jnp.float32)


def _finalize(pos_sq, neg_sq, epen, rpen):
    batch = pos_sq.shape[0] * pos_sq.shape[1]

    def body(pos_ref, neg_ref, epen_ref, rpen_ref, out_ref):
        p = jnp.sqrt(pos_ref[...])
        n = jnp.sqrt(neg_ref[...])
        loss = jnp.sum(jnp.maximum(p - n + MARGIN, 0.0)) / batch
        ent = jnp.sum(epen_ref[...]) / (4.0 * batch)
        rel = jnp.sum(rpen_ref[...]) / (2.0 * batch)
        out_ref[...] = jnp.full((1, 1), loss + C * (ent + rel), jnp.float32)

    return pl.pallas_call(
        body,
        out_shape=jax.ShapeDtypeStruct((1, 1), jnp.float32),
    )(pos_sq, neg_sq, epen, rpen)


def kernel(triple, corrupted_triple, entity_emb, relation_emb):
    h = triple[:, 0].astype(jnp.int32)
    r = triple[:, 1].astype(jnp.int32)
    t = triple[:, 2].astype(jnp.int32)
    hc = corrupted_triple[:, 0].astype(jnp.int32)
    rc = corrupted_triple[:, 1].astype(jnp.int32)
    tc = corrupted_triple[:, 2].astype(jnp.int32)
    batch = h.shape[0]
    hh = jnp.concatenate([h, hc])
    tt = jnp.concatenate([t, tc])
    rr = jnp.concatenate([r, rc])
    ent2 = _pack_pairs(entity_emb.T, PACK_W_ENT)
    rel2 = _pack_pairs(relation_emb.T, PACK_W_REL)
    hh2, ho = _pair_split(hh, PACK_W_ENT)
    tt2, to = _pair_split(tt, PACK_W_ENT)
    rr2, ro = _pair_split(rr, PACK_W_REL)
    dsq, epen, rpen = _sc_distances(
        2 * batch, ent2, rel2, hh2, tt2, rr2, ho, to, ro)
    pos_sq = dsq[:batch].reshape(128, -1)
    neg_sq = dsq[batch:].reshape(128, -1)
    out = _finalize(pos_sq, neg_sq, epen, rpen)
    return out[0, 0]


# speedup vs baseline: 3.5357x; 1.0898x over previous
import functools

import jax
import jax.numpy as jnp
from jax import lax
from jax.experimental import pallas as pl
from jax.experimental.pallas import tpu as pltpu
from jax.experimental.pallas import tpu_sc as plsc

DIM = 64
NCORES = 2
NSUB = 16
NW = NCORES * NSUB
CHUNK = 128
MARGIN = 1.0
C = 0.01


ENT_SH = 14
REL_SH = 10


@functools.partial(jax.jit, static_argnums=(0,))
def _sc_distances(bcat, ent2, rel2, idx3):
    per_w = bcat // NW
    n_chunks = per_w // CHUNK
    half = bcat // (2 * CHUNK)
    mesh = plsc.VectorSubcoreMesh(core_axis_name="c", subcore_axis_name="s")

    @functools.partial(
        pl.kernel,
        mesh=mesh,
        compiler_params=pltpu.CompilerParams(use_tc_tiling_on_sc=True),
        out_type=[
            jax.ShapeDtypeStruct((half, CHUNK), jnp.float32),
            jax.ShapeDtypeStruct((half, CHUNK), jnp.float32),
            jax.ShapeDtypeStruct((NW, 16), jnp.float32),
            jax.ShapeDtypeStruct((NW, 16), jnp.float32),
        ],
        scratch_types=(
            2 * [pltpu.VMEM((3, CHUNK), jnp.int32)]
            + 2 * [pltpu.VMEM((CHUNK,), jnp.int32)] * 3
            + 2 * [pltpu.VMEM((CHUNK, 2 * DIM), jnp.float32)] * 3
            + [
                pltpu.VMEM((CHUNK,), jnp.float32),
                pltpu.VMEM((16,), jnp.float32),
                pltpu.SemaphoreType.DMA,
                pltpu.SemaphoreType.DMA,
            ]
        ),
    )
    def k(ent_hbm, rel_hbm, idx3_hbm,
          pos_hbm, neg_hbm, epen_hbm, rpen_hbm,
          idx_v0, idx_v1,
          h_v0, t_v0, r_v0, h_v1, t_v1, r_v1,
          hrow0, trow0, rrow0, hrow1, trow1, rrow1,
          dbuf, penbuf, sem0, sem1):
        wid = lax.axis_index("s") * NCORES + lax.axis_index("c")
        base_w = wid * per_w
        lanes = lax.iota(jnp.int32, 16)
        bufs = [
            (idx_v0, h_v0, t_v0, r_v0, hrow0, trow0, rrow0, sem0),
            (idx_v1, h_v1, t_v1, r_v1, hrow1, trow1, rrow1, sem1),
        ]

        dnums = lax.GatherDimensionNumbers(
            offset_dims=(), collapsed_slice_dims=(0,), start_index_map=(0,))

        def shuf(x, idx):
            return lax.gather(
                x, idx[:, None], dimension_numbers=dnums, slice_sizes=(1,),
                mode=lax.GatherScatterMode.PROMISE_IN_BOUNDS)

        def issue(ci, buf):
            idx_v, h_v, t_v, r_v, hrow, trow, rrow, sem = buf
            cg = wid * n_chunks + ci
            pltpu.sync_copy(idx3_hbm.at[cg], idx_v)
            for kk, dst, sh_w, msk in (
                    (0, h_v, ENT_SH, (1 << (ENT_SH - 1)) - 1),
                    (1, t_v, ENT_SH, (1 << (ENT_SH - 1)) - 1),
                    (2, r_v, REL_SH, (1 << (REL_SH - 1)) - 1)):
                for b in range(CHUNK // 16):
                    e = idx_v[kk, pl.ds(b * 16, 16)]
                    dst[pl.ds(b * 16, 16)] = (
                        ((e >> sh_w) << (sh_w - 1)) | (e & msk))
            pltpu.async_copy(ent_hbm.at[h_v], hrow, sem)
            pltpu.async_copy(ent_hbm.at[t_v], trow, sem)
            pltpu.async_copy(rel_hbm.at[r_v], rrow, sem)

        def wait(buf):
            idx_v, h_v, t_v, r_v, hrow, trow, rrow, sem = buf
            pltpu.make_async_copy(ent_hbm.at[h_v], hrow, sem).wait()
            pltpu.make_async_copy(ent_hbm.at[t_v], trow, sem).wait()
            pltpu.make_async_copy(rel_hbm.at[r_v], rrow, sem).wait()

        def compute(ci, buf, accs):
            idx_v, h_v, t_v, r_v, hrow, trow, rrow, sem = buf
            cg = wid * n_chunks + ci

            def group_body(g, carry):
                ea, ra = carry
                acc_d = jnp.zeros((16,), jnp.float32)
                pv_h = ((idx_v[0, pl.ds(g * 16, 16)] >> (ENT_SH - 1))
                        & 1).astype(jnp.float32)
                pv_t = ((idx_v[1, pl.ds(g * 16, 16)] >> (ENT_SH - 1))
                        & 1).astype(jnp.float32)
                pv_r = ((idx_v[2, pl.ds(g * 16, 16)] >> (REL_SH - 1))
                        & 1).astype(jnp.float32)
                for jj in range(16):
                    j = g * 16 + jj
                    bidx = jnp.full((16,), jj, jnp.int32)
                    ph = shuf(pv_h, bidx)
                    pt = shuf(pv_t, bidx)
                    pr = shuf(pv_r, bidx)
                    sd = sh = st = sr = None
                    for q in range(DIM // 16):
                        hlo = hrow[j, pl.ds(q * 16, 16)]
                        hq = hlo + ph * (hrow[j, pl.ds(DIM + q * 16, 16)]
                                         - hlo)
                        rlo = rrow[j, pl.ds(q * 16, 16)]
                        rq = rlo + pr * (rrow[j, pl.ds(DIM + q * 16, 16)]
                                         - rlo)
                        tlo = trow[j, pl.ds(q * 16, 16)]
                        tq = tlo + pt * (trow[j, pl.ds(DIM + q * 16, 16)]
                                         - tlo)
                        d = hq + rq - tq
                        if q == 0:
                            sd, sh, st, sr = d * d, hq * hq, tq * tq, rq * rq
                        else:
                            sd = sd + d * d
                            sh = sh + hq * hq
                            st = st + tq * tq
                            sr = sr + rq * rq
                    csd = sd
                    for s in (8, 4, 2, 1):
                        csd = csd + shuf(csd, lanes ^ s)
                    for s in (8, 4):
                        sh = sh + shuf(sh, lanes ^ s)
                        st = st + shuf(st, lanes ^ s)
                        sr = sr + shuf(sr, lanes ^ s)
                    m = jnp.where(lanes < 4, sh,
                                  jnp.where(lanes < 8, st,
                                            jnp.where(lanes < 12, sr, 0.0)))
                    for s in (2, 1):
                        m = m + shuf(m, lanes ^ s)
                    pe = jnp.maximum(m - 1.0, 0.0)
                    acc_d = jnp.where(lanes == jj, csd, acc_d)
                    ea = ea + jnp.where(lanes < 8, pe, 0.0)
                    ra = ra + jnp.where((lanes >= 8) & (lanes < 12), pe, 0.0)
                dbuf[pl.ds(g * 16, 16)] = acc_d
                return (ea, ra)

            accs = lax.fori_loop(0, CHUNK // 16, group_body, accs)

            @pl.when(cg < half)
            def _():
                pltpu.sync_copy(dbuf, pos_hbm.at[cg])

            @pl.when(cg >= half)
            def _():
                pltpu.sync_copy(dbuf, neg_hbm.at[cg - half])

            return accs

        issue(0, bufs[0])
        n_super = n_chunks // 2

        def super_body(s, accs):
            c0 = 2 * s
            issue(c0 + 1, bufs[1])
            wait(bufs[0])
            accs = compute(c0, bufs[0], accs)

            @pl.when(s < n_super - 1)
            def _():
                issue(c0 + 2, bufs[0])

            wait(bufs[1])
            return compute(c0 + 1, bufs[1], accs)

        zero = jnp.zeros((16,), jnp.float32)
        ent_acc, rel_acc = lax.fori_loop(0, n_super, super_body, (zero, zero))
        penbuf[...] = ent_acc * 0.25
        pltpu.sync_copy(penbuf, epen_hbm.at[wid])
        penbuf[...] = rel_acc * 0.25
        pltpu.sync_copy(penbuf, rpen_hbm.at[wid])

    return k(ent2, rel2, idx3)


PACK_W_ENT = 16384
PACK_W_REL = 1024


def _pack_pairs(table_t, w):
    d, n = table_t.shape
    hw = w // 2
    grid = (n + w - 1) // w

    def body(in_ref, out_ref):
        x = in_ref[...]
        out_ref[...] = jnp.concatenate(
            [x[:, 0:hw], x[:, hw:w]], axis=0).T

    return pl.pallas_call(
        body,
        grid=(grid,),
        in_specs=[pl.BlockSpec((d, w), lambda i: (0, i))],
        out_specs=pl.BlockSpec((hw, 2 * d), lambda i: (i, 0)),
        out_shape=jax.ShapeDtypeStruct((grid * hw, 2 * d), jnp.float32),
    )(table_t)


def _finalize(pos_sq, neg_sq, epen, rpen):
    batch = pos_sq.shape[0] * pos_sq.shape[1]

    def body(pos_ref, neg_ref, epen_ref, rpen_ref, out_ref):
        p = jnp.sqrt(pos_ref[...])
        n = jnp.sqrt(neg_ref[...])
        loss = jnp.sum(jnp.maximum(p - n + MARGIN, 0.0)) / batch
        ent = jnp.sum(epen_ref[...]) / (4.0 * batch)
        rel = jnp.sum(rpen_ref[...]) / (2.0 * batch)
        out_ref[...] = jnp.full((1, 1), loss + C * (ent + rel), jnp.float32)

    return pl.pallas_call(
        body,
        out_shape=jax.ShapeDtypeStruct((1, 1), jnp.float32),
    )(pos_sq, neg_sq, epen, rpen)


def kernel(triple, corrupted_triple, entity_emb, relation_emb):
    h = triple[:, 0].astype(jnp.int32)
    r = triple[:, 1].astype(jnp.int32)
    t = triple[:, 2].astype(jnp.int32)
    hc = corrupted_triple[:, 0].astype(jnp.int32)
    rc = corrupted_triple[:, 1].astype(jnp.int32)
    tc = corrupted_triple[:, 2].astype(jnp.int32)
    batch = h.shape[0]
    hh = jnp.concatenate([h, hc])
    tt = jnp.concatenate([t, tc])
    rr = jnp.concatenate([r, rc])
    ent2 = _pack_pairs(entity_emb.T, PACK_W_ENT)
    rel2 = _pack_pairs(relation_emb.T, PACK_W_REL)
    idx3 = jnp.stack([hh.reshape(-1, CHUNK), tt.reshape(-1, CHUNK),
                      rr.reshape(-1, CHUNK)], axis=1)
    pos_sq, neg_sq, epen, rpen = _sc_distances(
        2 * batch, ent2, rel2, idx3)
    out = _finalize(pos_sq, neg_sq, epen, rpen)
    return out[0, 0]


# speedup vs baseline: 3.6099x; 1.0210x over previous
import functools

import jax
import jax.numpy as jnp
from jax import lax
from jax.experimental import pallas as pl
from jax.experimental.pallas import tpu as pltpu
from jax.experimental.pallas import tpu_sc as plsc

DIM = 64
NCORES = 2
NSUB = 16
NW = NCORES * NSUB
CHUNK = 128
MARGIN = 1.0
C = 0.01


ENT_SH = 15
REL_SH = 10


@functools.partial(jax.jit, static_argnums=(0,))
def _sc_distances(bcat, ent2, rel2, idx3):
    per_w = bcat // NW
    n_chunks = per_w // CHUNK
    half = bcat // (2 * CHUNK)
    mesh = plsc.VectorSubcoreMesh(core_axis_name="c", subcore_axis_name="s")

    @functools.partial(
        pl.kernel,
        mesh=mesh,
        compiler_params=pltpu.CompilerParams(use_tc_tiling_on_sc=True),
        out_type=[
            jax.ShapeDtypeStruct((half, CHUNK), jnp.float32),
            jax.ShapeDtypeStruct((half, CHUNK), jnp.float32),
            jax.ShapeDtypeStruct((NW, 16), jnp.float32),
            jax.ShapeDtypeStruct((NW, 16), jnp.float32),
        ],
        scratch_types=(
            2 * [pltpu.VMEM((3, CHUNK), jnp.int32)]
            + 2 * [pltpu.VMEM((CHUNK,), jnp.int32)] * 3
            + 2 * [pltpu.VMEM((CHUNK, 2 * DIM), jnp.float32)] * 3
            + [
                pltpu.VMEM((CHUNK,), jnp.float32),
                pltpu.VMEM((16,), jnp.float32),
                pltpu.SemaphoreType.DMA,
                pltpu.SemaphoreType.DMA,
            ]
        ),
    )
    def k(ent_hbm, rel_hbm, idx3_hbm,
          pos_hbm, neg_hbm, epen_hbm, rpen_hbm,
          idx_v0, idx_v1,
          h_v0, t_v0, r_v0, h_v1, t_v1, r_v1,
          hrow0, trow0, rrow0, hrow1, trow1, rrow1,
          dbuf, penbuf, sem0, sem1):
        wid = lax.axis_index("s") * NCORES + lax.axis_index("c")
        base_w = wid * per_w
        lanes = lax.iota(jnp.int32, 16)
        bufs = [
            (idx_v0, h_v0, t_v0, r_v0, hrow0, trow0, rrow0, sem0),
            (idx_v1, h_v1, t_v1, r_v1, hrow1, trow1, rrow1, sem1),
        ]

        dnums = lax.GatherDimensionNumbers(
            offset_dims=(), collapsed_slice_dims=(0,), start_index_map=(0,))

        def shuf(x, idx):
            return lax.gather(
                x, idx[:, None], dimension_numbers=dnums, slice_sizes=(1,),
                mode=lax.GatherScatterMode.PROMISE_IN_BOUNDS)

        def issue(ci, buf):
            idx_v, h_v, t_v, r_v, hrow, trow, rrow, sem = buf
            cg = wid * n_chunks + ci
            pltpu.sync_copy(idx3_hbm.at[cg], idx_v)
            for kk, dst, sh_w, msk in (
                    (0, h_v, ENT_SH, (1 << (ENT_SH - 1)) - 1),
                    (1, t_v, ENT_SH, (1 << (ENT_SH - 1)) - 1),
                    (2, r_v, REL_SH, (1 << (REL_SH - 1)) - 1)):
                for b in range(CHUNK // 16):
                    e = idx_v[kk, pl.ds(b * 16, 16)]
                    dst[pl.ds(b * 16, 16)] = (
                        ((e >> sh_w) << (sh_w - 1)) | (e & msk))
            pltpu.async_copy(ent_hbm.at[h_v], hrow, sem)
            pltpu.async_copy(ent_hbm.at[t_v], trow, sem)
            pltpu.async_copy(rel_hbm.at[r_v], rrow, sem)

        def wait(buf):
            idx_v, h_v, t_v, r_v, hrow, trow, rrow, sem = buf
            pltpu.make_async_copy(ent_hbm.at[h_v], hrow, sem).wait()
            pltpu.make_async_copy(ent_hbm.at[t_v], trow, sem).wait()
            pltpu.make_async_copy(rel_hbm.at[r_v], rrow, sem).wait()

        def compute(ci, buf, accs):
            idx_v, h_v, t_v, r_v, hrow, trow, rrow, sem = buf
            cg = wid * n_chunks + ci

            def group_body(g, carry):
                ea, ra = carry
                acc_d = jnp.zeros((16,), jnp.float32)
                pv_h = ((idx_v[0, pl.ds(g * 16, 16)] >> (ENT_SH - 1))
                        & 1).astype(jnp.float32)
                pv_t = ((idx_v[1, pl.ds(g * 16, 16)] >> (ENT_SH - 1))
                        & 1).astype(jnp.float32)
                pv_r = ((idx_v[2, pl.ds(g * 16, 16)] >> (REL_SH - 1))
                        & 1).astype(jnp.float32)
                for jj in range(16):
                    j = g * 16 + jj
                    bidx = jnp.full((16,), jj, jnp.int32)
                    ph = shuf(pv_h, bidx)
                    pt = shuf(pv_t, bidx)
                    pr = shuf(pv_r, bidx)
                    sd = sh = st = sr = None
                    for q in range(DIM // 16):
                        hlo = hrow[j, pl.ds(q * 16, 16)]
                        hq = hlo + ph * (hrow[j, pl.ds(DIM + q * 16, 16)]
                                         - hlo)
                        rlo = rrow[j, pl.ds(q * 16, 16)]
                        rq = rlo + pr * (rrow[j, pl.ds(DIM + q * 16, 16)]
                                         - rlo)
                        tlo = trow[j, pl.ds(q * 16, 16)]
                        tq = tlo + pt * (trow[j, pl.ds(DIM + q * 16, 16)]
                                         - tlo)
                        d = hq + rq - tq
                        if q == 0:
                            sd, sh, st, sr = d * d, hq * hq, tq * tq, rq * rq
                        else:
                            sd = sd + d * d
                            sh = sh + hq * hq
                            st = st + tq * tq
                            sr = sr + rq * rq
                    csd = sd
                    for s in (8, 4, 2, 1):
                        csd = csd + shuf(csd, lanes ^ s)
                    for s in (8, 4):
                        sh = sh + shuf(sh, lanes ^ s)
                        st = st + shuf(st, lanes ^ s)
                        sr = sr + shuf(sr, lanes ^ s)
                    m = jnp.where(lanes < 4, sh,
                                  jnp.where(lanes < 8, st,
                                            jnp.where(lanes < 12, sr, 0.0)))
                    for s in (2, 1):
                        m = m + shuf(m, lanes ^ s)
                    pe = jnp.maximum(m - 1.0, 0.0)
                    acc_d = jnp.where(lanes == jj, csd, acc_d)
                    ea = ea + jnp.where(lanes < 8, pe, 0.0)
                    ra = ra + jnp.where((lanes >= 8) & (lanes < 12), pe, 0.0)
                dbuf[pl.ds(g * 16, 16)] = acc_d
                return (ea, ra)

            accs = lax.fori_loop(0, CHUNK // 16, group_body, accs)

            @pl.when(cg < half)
            def _():
                pltpu.sync_copy(dbuf, pos_hbm.at[cg])

            @pl.when(cg >= half)
            def _():
                pltpu.sync_copy(dbuf, neg_hbm.at[cg - half])

            return accs

        issue(0, bufs[0])
        n_super = n_chunks // 2

        def super_body(s, accs):
            c0 = 2 * s
            issue(c0 + 1, bufs[1])
            wait(bufs[0])
            accs = compute(c0, bufs[0], accs)

            @pl.when(s < n_super - 1)
            def _():
                issue(c0 + 2, bufs[0])

            wait(bufs[1])
            return compute(c0 + 1, bufs[1], accs)

        zero = jnp.zeros((16,), jnp.float32)
        ent_acc, rel_acc = lax.fori_loop(0, n_super, super_body, (zero, zero))
        penbuf[...] = ent_acc * 0.25
        pltpu.sync_copy(penbuf, epen_hbm.at[wid])
        penbuf[...] = rel_acc * 0.25
        pltpu.sync_copy(penbuf, rpen_hbm.at[wid])

    return k(ent2, rel2, idx3)


PACK_W_ENT = 32768
PACK_W_REL = 1024


def _pack_pairs(table_t, w):
    d, n = table_t.shape
    hw = w // 2
    grid = (n + w - 1) // w

    def body(in_ref, out_ref):
        x = in_ref[...]
        out_ref[...] = jnp.concatenate(
            [x[:, 0:hw], x[:, hw:w]], axis=0).T

    return pl.pallas_call(
        body,
        grid=(grid,),
        in_specs=[pl.BlockSpec((d, w), lambda i: (0, i))],
        out_specs=pl.BlockSpec((hw, 2 * d), lambda i: (i, 0)),
        out_shape=jax.ShapeDtypeStruct((grid * hw, 2 * d), jnp.float32),
    )(table_t)


def _finalize(pos_sq, neg_sq, epen, rpen):
    batch = pos_sq.shape[0] * pos_sq.shape[1]

    def body(pos_ref, neg_ref, epen_ref, rpen_ref, out_ref):
        p = jnp.sqrt(pos_ref[...])
        n = jnp.sqrt(neg_ref[...])
        loss = jnp.sum(jnp.maximum(p - n + MARGIN, 0.0)) / batch
        ent = jnp.sum(epen_ref[...]) / (4.0 * batch)
        rel = jnp.sum(rpen_ref[...]) / (2.0 * batch)
        out_ref[...] = jnp.full((1, 1), loss + C * (ent + rel), jnp.float32)

    return pl.pallas_call(
        body,
        out_shape=jax.ShapeDtypeStruct((1, 1), jnp.float32),
    )(pos_sq, neg_sq, epen, rpen)


def kernel(triple, corrupted_triple, entity_emb, relation_emb):
    h = triple[:, 0].astype(jnp.int32)
    r = triple[:, 1].astype(jnp.int32)
    t = triple[:, 2].astype(jnp.int32)
    hc = corrupted_triple[:, 0].astype(jnp.int32)
    rc = corrupted_triple[:, 1].astype(jnp.int32)
    tc = corrupted_triple[:, 2].astype(jnp.int32)
    batch = h.shape[0]
    hh = jnp.concatenate([h, hc])
    tt = jnp.concatenate([t, tc])
    rr = jnp.concatenate([r, rc])
    ent2 = _pack_pairs(entity_emb.T, PACK_W_ENT)
    rel2 = _pack_pairs(relation_emb.T, PACK_W_REL)
    idx3 = jnp.stack([hh.reshape(-1, CHUNK), tt.reshape(-1, CHUNK),
                      rr.reshape(-1, CHUNK)], axis=1)
    pos_sq, neg_sq, epen, rpen = _sc_distances(
        2 * batch, ent2, rel2, idx3)
    out = _finalize(pos_sq, neg_sq, epen, rpen)
    return out[0, 0]
